# baseline (speedup 1.0000x reference)
import functools

import jax
import jax.numpy as jnp
from jax import lax
from jax.experimental import pallas as pl
from jax.experimental.pallas import tpu as pltpu
from jax.experimental.pallas import tpu_sc as plsc

_N = 10000
_G = 32
_GP = _G + 2
_C = 256
_NOFF = 27
_NW = 32
_PPW = 320
_NPAD = _NW * _PPW
_NZ = _NPAD - _N
_CHUNK = 4
_NCHUNK = _PPW // _CHUNK
_ROWS = _NOFF * _CHUNK
_ROWS_PAD = 112
_TBL = 39312
_L = 16


def _mm_body(f_ref, w_ref, y_ref):
    f = f_ref[...]
    for o in range(_NOFF):
        y_ref[o] = jnp.dot(
            f, w_ref[o], preferred_element_type=jnp.float32
        ).astype(jnp.bfloat16)


def _compute_y(fpad, w27):
    mtile = 256
    return pl.pallas_call(
        _mm_body,
        grid=(_NPAD // mtile,),
        in_specs=[
            pl.BlockSpec((mtile, _C), lambda i: (i, 0)),
            pl.BlockSpec((_NOFF, _C, _C), lambda i: (0, 0, 0)),
        ],
        out_specs=pl.BlockSpec((_NOFF, mtile, _C), lambda i: (0, i, 0)),
        out_shape=jax.ShapeDtypeStruct((_NOFF, _NPAD, _C), jnp.bfloat16),
    )(fpad, w27)


_DELTAS = []
for _dx in range(-1, 2):
    for _dy in range(-1, 2):
        for _dz in range(-1, 2):
            _DELTAS.append(_dx * (_GP * _GP) + _dy * _GP + _dz)


def _sc_gather_reduce(y_flat, lin_pad):
    mesh = plsc.VectorSubcoreMesh(core_axis_name="c", subcore_axis_name="s")

    @functools.partial(
        pl.kernel,
        mesh=mesh,
        out_type=jax.ShapeDtypeStruct((_NPAD, _C), jnp.float32),
        compiler_params=pltpu.CompilerParams(needs_layout_passes=False),
        scratch_types=[
            pltpu.VMEM((_TBL,), jnp.int32),
            pltpu.VMEM((_NPAD,), jnp.int32),
            pltpu.VMEM((_NCHUNK * _ROWS_PAD,), jnp.int32),
            pltpu.VMEM((2, _ROWS_PAD, _C // 2), jnp.int32),
            pltpu.VMEM((2, _CHUNK, _C), jnp.float32),
            pltpu.SemaphoreType.DMA((2,)),
            pltpu.SemaphoreType.DMA((2,)),
        ],
    )
    def k(y_hbm, lin_hbm, out_hbm, tbl, lin_v, idx_v, stag, acc, gsem, osem):
        cid = lax.axis_index("c")
        sid = lax.axis_index("s")
        wid = sid * 2 + cid
        base = wid * _PPW
        pltpu.sync_copy(lin_hbm, lin_v)

        lane = lax.iota(jnp.int32, _L)

        def zb(i, _):
            tbl[pl.ds(i * _L, _L)] = jnp.full((_L,), -1, jnp.int32)
            return 0

        lax.fori_loop(0, _TBL // _L, zb, 0)

        def sb(q, _):
            lv = lin_v[pl.ds(q * _L, _L)]
            ids = q * _L + lane
            plsc.store_scatter(tbl, [lv], ids)
            return 0

        lax.fori_loop(0, _NPAD // _L, sb, 0)

        def gb(g, _):
            pid = g * _L + lane
            lv = lin_v[pl.ds(base + g * _L, _L)]
            ci = lax.shift_right_logical(pid, 2)
            prem = jnp.bitwise_and(pid, 3)
            pos_base = ci * _ROWS_PAD + prem
            for o in range(_NOFF):
                nl = jnp.maximum(lv + _DELTAS[o], 0)
                t = plsc.load_gather(tbl, [nl])
                spread = o * _NPAD + _N + jnp.remainder(pid + o * 9, _NZ)
                gi = jnp.where(t >= 0, o * _NPAD + t, spread)
                plsc.store_scatter(idx_v, [pos_base + o * _CHUNK], gi)
            return 0

        lax.fori_loop(0, _PPW // _L, gb, 0)

        def pb(f, _):
            cis = f * _L + lane
            for kk in range(_ROWS_PAD - _ROWS):
                pos = cis * _ROWS_PAD + _ROWS + kk
                val = kk * _NPAD + _N + jnp.remainder(cis * 4 + kk * 61, _NZ)
                plsc.store_scatter(idx_v, [pos], val)
            return 0

        lax.fori_loop(0, _NCHUNK // _L, pb, 0)

        def fire(ci, b):
            pltpu.async_copy(
                y_hbm.at[idx_v.at[pl.ds(ci * _ROWS_PAD, _ROWS_PAD)]],
                stag.at[b],
                gsem.at[b],
            )

        def drain_gather(b):
            pltpu.make_async_copy(
                y_hbm.at[pl.ds(0, _ROWS_PAD)], stag.at[b], gsem.at[b]
            ).wait()

        def reduce(b):
            def cbody(c, _):
                def pbody(p, _):
                    ae, ao = plsc.unpack(
                        plsc.bitcast(
                            stag[b, p, pl.ds(c * _L, _L)], jnp.bfloat16
                        ),
                        format=plsc.PackFormat.INTERLEAVED,
                    )
                    for o in range(1, _NOFF):
                        e2, o2 = plsc.unpack(
                            plsc.bitcast(
                                stag[b, o * _CHUNK + p, pl.ds(c * _L, _L)],
                                jnp.bfloat16,
                            ),
                            format=plsc.PackFormat.INTERLEAVED,
                        )
                        ae = ae + e2
                        ao = ao + o2
                    acc[b, p, pl.ds(c * 2 * _L, _L)] = ae
                    acc[b, p, pl.ds(c * 2 * _L + _L, _L)] = ao
                    return 0

                lax.fori_loop(0, _CHUNK, pbody, 0)
                return 0

            lax.fori_loop(0, _C // (2 * _L), cbody, 0)

        fire(0, 0)
        fire(1, 1)

        def outer(t, _):
            for b in (0, 1):
                ci = t * 2 + b
                drain_gather(b)

                @pl.when(t > 0)
                def _drain_out():
                    pltpu.make_async_copy(
                        y_hbm.at[pl.ds(0, _CHUNK)], acc.at[b], osem.at[b]
                    ).wait()

                reduce(b)

                @pl.when(t < _NCHUNK // 2 - 1)
                def _fire_next():
                    fire(ci + 2, b)

                pltpu.async_copy(
                    acc.at[b],
                    out_hbm.at[pl.ds(base + ci * _CHUNK, _CHUNK)],
                    osem.at[b],
                )
            return 0

        lax.fori_loop(0, _NCHUNK // 2, outer, 0)
        for b in (0, 1):
            pltpu.make_async_copy(
                y_hbm.at[pl.ds(0, _CHUNK)], acc.at[b], osem.at[b]
            ).wait()

    return k(y_flat, lin_pad)


def kernel(features, inp_positions, W, voxel_size):
    v = jnp.floor(inp_positions / voxel_size).astype(jnp.int32)
    lin = (v[:, 0] + 1) * (_GP * _GP) + (v[:, 1] + 1) * _GP + (v[:, 2] + 1)
    lin_pad = jnp.pad(lin, (0, _NPAD - _N))

    fpad = jnp.pad(features, ((0, _NPAD - _N), (0, 0))).astype(jnp.bfloat16)
    w27 = W.reshape(_NOFF, _C, _C)
    blk = jnp.arange(_C).reshape(_C // 32, 32)
    perm = jnp.stack([blk[:, :16], blk[:, 16:]], axis=2).reshape(_C)
    w27p = w27[:, :, perm].astype(jnp.bfloat16)
    y = _compute_y(fpad, w27p).reshape(_NOFF * _NPAD, _C)
    y = lax.bitcast_convert_type(
        y.reshape(_NOFF * _NPAD, _C // 2, 2), jnp.int32
    )
    out = _sc_gather_reduce(y, lin_pad)
    return out[:_N]


# speedup vs baseline: 7.0010x; 7.0010x over previous
import functools

import jax
import jax.numpy as jnp
from jax import lax
from jax.experimental import pallas as pl
from jax.experimental.pallas import tpu as pltpu
from jax.experimental.pallas import tpu_sc as plsc

_N = 10000
_G = 32
_GP = _G + 2
_C = 256
_NOFF = 27
_NW = 32
_PPW = 320
_NPAD = _NW * _PPW
_NZ = _NPAD - _N
_CHUNK = 4
_NCHUNK = _PPW // _CHUNK
_ROWS = _NOFF * _CHUNK
_ROWS_PAD = 112
_TBL = 39312
_L = 16


def _rne16(x):
    xi = lax.bitcast_convert_type(x, jnp.int32)
    lsb = jnp.bitwise_and(lax.shift_right_logical(xi, 16), 1)
    return lax.shift_right_logical(xi + 0x7FFF + lsb, 16)


def _mm_body(f_ref, w_ref, y_ref):
    f = f_ref[...]
    for o in range(_NOFF):
        y = jnp.dot(f, w_ref[o], preferred_element_type=jnp.float32)
        lo = _rne16(y[:, : _C // 2])
        hi = _rne16(y[:, _C // 2 :])
        y_ref[o] = jnp.bitwise_or(lo, lax.shift_left(hi, 16))


def _compute_y(fpad, w27):
    mtile = 256
    return pl.pallas_call(
        _mm_body,
        grid=(_NPAD // mtile,),
        in_specs=[
            pl.BlockSpec((mtile, _C), lambda i: (i, 0)),
            pl.BlockSpec((_NOFF, _C, _C), lambda i: (0, 0, 0)),
        ],
        out_specs=pl.BlockSpec((_NOFF, mtile, _C // 2), lambda i: (0, i, 0)),
        out_shape=jax.ShapeDtypeStruct((_NOFF, _NPAD, _C // 2), jnp.int32),
    )(fpad, w27)


_DELTAS = []
for _dx in range(-1, 2):
    for _dy in range(-1, 2):
        for _dz in range(-1, 2):
            _DELTAS.append(_dx * (_GP * _GP) + _dy * _GP + _dz)


def _sc_gather_reduce(y_flat, lin_pad):
    mesh = plsc.VectorSubcoreMesh(core_axis_name="c", subcore_axis_name="s")

    @functools.partial(
        pl.kernel,
        mesh=mesh,
        out_type=jax.ShapeDtypeStruct((_NPAD, _C), jnp.float32),
        compiler_params=pltpu.CompilerParams(needs_layout_passes=False),
        scratch_types=[
            pltpu.VMEM((_TBL,), jnp.int32),
            pltpu.VMEM((_NPAD,), jnp.int32),
            pltpu.VMEM((_NCHUNK * _ROWS_PAD,), jnp.int32),
            pltpu.VMEM((2, _ROWS_PAD, _C // 2), jnp.int32),
            pltpu.VMEM((2, _CHUNK, _C), jnp.float32),
            pltpu.SemaphoreType.DMA((2,)),
            pltpu.SemaphoreType.DMA((2,)),
        ],
    )
    def k(y_hbm, lin_hbm, out_hbm, tbl, lin_v, idx_v, stag, acc, gsem, osem):
        cid = lax.axis_index("c")
        sid = lax.axis_index("s")
        wid = sid * 2 + cid
        base = wid * _PPW
        pltpu.sync_copy(lin_hbm, lin_v)

        lane = lax.iota(jnp.int32, _L)

        def zb(i, _):
            tbl[pl.ds(i * _L, _L)] = jnp.full((_L,), -1, jnp.int32)
            return 0

        lax.fori_loop(0, _TBL // _L, zb, 0)

        def sb(q, _):
            lv = lin_v[pl.ds(q * _L, _L)]
            ids = q * _L + lane
            plsc.store_scatter(tbl, [lv], ids)
            return 0

        lax.fori_loop(0, _NPAD // _L, sb, 0)

        def gb(g, _):
            pid = g * _L + lane
            lv = lin_v[pl.ds(base + g * _L, _L)]
            ci = lax.shift_right_logical(pid, 2)
            prem = jnp.bitwise_and(pid, 3)
            pos_base = ci * _ROWS_PAD + prem
            for o in range(_NOFF):
                nl = jnp.maximum(lv + _DELTAS[o], 0)
                t = plsc.load_gather(tbl, [nl])
                spread = o * _NPAD + _N + jnp.remainder(pid + o * 9, _NZ)
                gi = jnp.where(t >= 0, o * _NPAD + t, spread)
                plsc.store_scatter(idx_v, [pos_base + o * _CHUNK], gi)
            return 0

        lax.fori_loop(0, _PPW // _L, gb, 0)

        def pb(f, _):
            cis = f * _L + lane
            for kk in range(_ROWS_PAD - _ROWS):
                pos = cis * _ROWS_PAD + _ROWS + kk
                val = kk * _NPAD + _N + jnp.remainder(cis * 4 + kk * 61, _NZ)
                plsc.store_scatter(idx_v, [pos], val)
            return 0

        lax.fori_loop(0, _NCHUNK // _L, pb, 0)

        def fire(ci, b):
            pltpu.async_copy(
                y_hbm.at[idx_v.at[pl.ds(ci * _ROWS_PAD, _ROWS_PAD)]],
                stag.at[b],
                gsem.at[b],
            )

        def drain_gather(b):
            pltpu.make_async_copy(
                y_hbm.at[pl.ds(0, _ROWS_PAD)], stag.at[b], gsem.at[b]
            ).wait()

        def reduce(b):
            def cbody(c, _):
                def pbody(p, _):
                    ae, ao = plsc.unpack(
                        plsc.bitcast(
                            stag[b, p, pl.ds(c * _L, _L)], jnp.bfloat16
                        ),
                        format=plsc.PackFormat.INTERLEAVED,
                    )
                    for o in range(1, _NOFF):
                        e2, o2 = plsc.unpack(
                            plsc.bitcast(
                                stag[b, o * _CHUNK + p, pl.ds(c * _L, _L)],
                                jnp.bfloat16,
                            ),
                            format=plsc.PackFormat.INTERLEAVED,
                        )
                        ae = ae + e2
                        ao = ao + o2
                    acc[b, p, pl.ds(c * _L, _L)] = ae
                    acc[b, p, pl.ds(_C // 2 + c * _L, _L)] = ao
                    return 0

                lax.fori_loop(0, _CHUNK, pbody, 0)
                return 0

            lax.fori_loop(0, _C // (2 * _L), cbody, 0)

        fire(0, 0)
        fire(1, 1)

        def outer(t, _):
            for b in (0, 1):
                ci = t * 2 + b
                drain_gather(b)

                @pl.when(t > 0)
                def _drain_out():
                    pltpu.make_async_copy(
                        y_hbm.at[pl.ds(0, _CHUNK)], acc.at[b], osem.at[b]
                    ).wait()

                reduce(b)

                @pl.when(t < _NCHUNK // 2 - 1)
                def _fire_next():
                    fire(ci + 2, b)

                pltpu.async_copy(
                    acc.at[b],
                    out_hbm.at[pl.ds(base + ci * _CHUNK, _CHUNK)],
                    osem.at[b],
                )
            return 0

        lax.fori_loop(0, _NCHUNK // 2, outer, 0)
        for b in (0, 1):
            pltpu.make_async_copy(
                y_hbm.at[pl.ds(0, _CHUNK)], acc.at[b], osem.at[b]
            ).wait()

    return k(y_flat, lin_pad)


def kernel(features, inp_positions, W, voxel_size):
    v = jnp.floor(inp_positions / voxel_size).astype(jnp.int32)
    lin = (v[:, 0] + 1) * (_GP * _GP) + (v[:, 1] + 1) * _GP + (v[:, 2] + 1)
    lin_pad = jnp.pad(lin, (0, _NPAD - _N))

    fpad = jnp.pad(features, ((0, _NPAD - _N), (0, 0))).astype(jnp.bfloat16)
    w27 = W.reshape(_NOFF, _C, _C).astype(jnp.bfloat16)
    y = _compute_y(fpad, w27).reshape(_NOFF * _NPAD, _C // 2)
    out = _sc_gather_reduce(y, lin_pad)
    return out[:_N]


# speedup vs baseline: 8.0277x; 1.1467x over previous
import functools

import jax
import jax.numpy as jnp
from jax import lax
from jax.experimental import pallas as pl
from jax.experimental.pallas import tpu as pltpu
from jax.experimental.pallas import tpu_sc as plsc

_N = 10000
_G = 32
_GP = _G + 2
_C = 256
_NOFF = 27
_NW = 32
_PPW = 320
_NPAD = _NW * _PPW
_NZ = _NPAD - _N
_CHUNK = 4
_NCHUNK = _PPW // _CHUNK
_ROWS = _NOFF * _CHUNK
_ROWS_PAD = 112
_TBL = 39312
_L = 16
_NBUF = 4


def _rne16(x):
    xi = lax.bitcast_convert_type(x, jnp.int32)
    return lax.shift_right_logical(xi + 0x8000, 16)


def _mm_body(f_ref, w_ref, y_ref):
    f = f_ref[...]
    for o in range(_NOFF):
        y = jnp.dot(f, w_ref[o], preferred_element_type=jnp.float32)
        lo = _rne16(y[:, : _C // 2])
        hi = _rne16(y[:, _C // 2 :])
        y_ref[o] = jnp.bitwise_or(lo, lax.shift_left(hi, 16))


def _compute_y(fpad, w27):
    mtile = 256
    return pl.pallas_call(
        _mm_body,
        grid=(_NPAD // mtile,),
        in_specs=[
            pl.BlockSpec((mtile, _C), lambda i: (i, 0)),
            pl.BlockSpec((_NOFF, _C, _C), lambda i: (0, 0, 0)),
        ],
        out_specs=pl.BlockSpec((_NOFF, mtile, _C // 2), lambda i: (0, i, 0)),
        out_shape=jax.ShapeDtypeStruct((_NOFF, _NPAD, _C // 2), jnp.int32),
    )(fpad, w27)


_DELTAS = []
for _dx in range(-1, 2):
    for _dy in range(-1, 2):
        for _dz in range(-1, 2):
            _DELTAS.append(_dx * (_GP * _GP) + _dy * _GP + _dz)


def _sc_gather_reduce(y_flat, lin_pad):
    mesh = plsc.VectorSubcoreMesh(core_axis_name="c", subcore_axis_name="s")

    @functools.partial(
        pl.kernel,
        mesh=mesh,
        out_type=jax.ShapeDtypeStruct((_NPAD, _C), jnp.float32),
        compiler_params=pltpu.CompilerParams(needs_layout_passes=False),
        scratch_types=[
            pltpu.VMEM((_TBL,), jnp.int32),
            pltpu.VMEM((_NPAD,), jnp.int32),
            pltpu.VMEM((_NCHUNK * _ROWS_PAD,), jnp.int32),
            pltpu.VMEM((_NBUF, _ROWS_PAD, _C // 2), jnp.int32),
            pltpu.VMEM((_NBUF, _CHUNK, _C), jnp.float32),
            pltpu.SemaphoreType.DMA((_NBUF,)),
            pltpu.SemaphoreType.DMA((_NBUF,)),
        ],
    )
    def k(y_hbm, lin_hbm, out_hbm, tbl, lin_v, idx_v, stag, acc, gsem, osem):
        cid = lax.axis_index("c")
        sid = lax.axis_index("s")
        wid = sid * 2 + cid
        base = wid * _PPW
        pltpu.sync_copy(lin_hbm, lin_v)

        lane = lax.iota(jnp.int32, _L)

        def zb(i, _):
            tbl[pl.ds(i * _L, _L)] = jnp.full((_L,), -1, jnp.int32)
            return 0

        lax.fori_loop(0, _TBL // _L, zb, 0)

        def sb(q, _):
            lv = lin_v[pl.ds(q * _L, _L)]
            ids = q * _L + lane
            plsc.store_scatter(tbl, [lv], ids)
            return 0

        lax.fori_loop(0, _NPAD // _L, sb, 0)

        def gb(g, _):
            pid = g * _L + lane
            lv = lin_v[pl.ds(base + g * _L, _L)]
            ci = lax.shift_right_logical(pid, 2)
            prem = jnp.bitwise_and(pid, 3)
            pos_base = ci * _ROWS_PAD + prem
            for o in range(_NOFF):
                nl = jnp.maximum(lv + _DELTAS[o], 0)
                t = plsc.load_gather(tbl, [nl])
                spread = o * _NPAD + _N + jnp.remainder(pid + o * 9, _NZ)
                gi = jnp.where(t >= 0, o * _NPAD + t, spread)
                plsc.store_scatter(idx_v, [pos_base + o * _CHUNK], gi)
            return 0

        lax.fori_loop(0, _PPW // _L, gb, 0)

        def pb(f, _):
            cis = f * _L + lane
            for kk in range(_ROWS_PAD - _ROWS):
                pos = cis * _ROWS_PAD + _ROWS + kk
                val = kk * _NPAD + _N + jnp.remainder(cis * 4 + kk * 61, _NZ)
                plsc.store_scatter(idx_v, [pos], val)
            return 0

        lax.fori_loop(0, _NCHUNK // _L, pb, 0)

        def fire(ci, b):
            pltpu.async_copy(
                y_hbm.at[idx_v.at[pl.ds(ci * _ROWS_PAD, _ROWS_PAD)]],
                stag.at[b],
                gsem.at[b],
            )

        def drain_gather(b):
            pltpu.make_async_copy(
                y_hbm.at[pl.ds(0, _ROWS_PAD)], stag.at[b], gsem.at[b]
            ).wait()

        def reduce(b):
            def cbody(c, _):
                def pbody(p, _):
                    ae, ao = plsc.unpack(
                        plsc.bitcast(
                            stag[b, p, pl.ds(c * _L, _L)], jnp.bfloat16
                        ),
                        format=plsc.PackFormat.INTERLEAVED,
                    )
                    for o in range(1, _NOFF):
                        e2, o2 = plsc.unpack(
                            plsc.bitcast(
                                stag[b, o * _CHUNK + p, pl.ds(c * _L, _L)],
                                jnp.bfloat16,
                            ),
                            format=plsc.PackFormat.INTERLEAVED,
                        )
                        ae = ae + e2
                        ao = ao + o2
                    acc[b, p, pl.ds(c * _L, _L)] = ae
                    acc[b, p, pl.ds(_C // 2 + c * _L, _L)] = ao
                    return 0

                lax.fori_loop(0, _CHUNK, pbody, 0)
                return 0

            lax.fori_loop(0, _C // (2 * _L), cbody, 0)

        for b in range(_NBUF):
            fire(b, b)

        def outer(t, _):
            for b in range(_NBUF):
                ci = t * _NBUF + b
                drain_gather(b)

                @pl.when(t > 0)
                def _drain_out():
                    pltpu.make_async_copy(
                        y_hbm.at[pl.ds(0, _CHUNK)], acc.at[b], osem.at[b]
                    ).wait()

                reduce(b)

                @pl.when(t < _NCHUNK // _NBUF - 1)
                def _fire_next():
                    fire(ci + _NBUF, b)

                pltpu.async_copy(
                    acc.at[b],
                    out_hbm.at[pl.ds(base + ci * _CHUNK, _CHUNK)],
                    osem.at[b],
                )
            return 0

        lax.fori_loop(0, _NCHUNK // _NBUF, outer, 0)
        for b in range(_NBUF):
            pltpu.make_async_copy(
                y_hbm.at[pl.ds(0, _CHUNK)], acc.at[b], osem.at[b]
            ).wait()

    return k(y_flat, lin_pad)


def kernel(features, inp_positions, W, voxel_size):
    v = jnp.floor(inp_positions / voxel_size).astype(jnp.int32)
    lin = (v[:, 0] + 1) * (_GP * _GP) + (v[:, 1] + 1) * _GP + (v[:, 2] + 1)
    lin_pad = jnp.pad(lin, (0, _NPAD - _N))

    fpad = jnp.pad(features, ((0, _NPAD - _N), (0, 0))).astype(jnp.bfloat16)
    w27 = W.reshape(_NOFF, _C, _C).astype(jnp.bfloat16)
    y = _compute_y(fpad, w27).reshape(_NOFF * _NPAD, _C // 2)
    out = _sc_gather_reduce(y, lin_pad)
    return out[:_N]


# speedup vs baseline: 8.4957x; 1.0583x over previous
import functools

import jax
import jax.numpy as jnp
from jax import lax
from jax.experimental import pallas as pl
from jax.experimental.pallas import tpu as pltpu
from jax.experimental.pallas import tpu_sc as plsc

_N = 10000
_G = 32
_GP = _G + 2
_C = 256
_NOFF = 27
_NW = 32
_PPW = 320
_NPAD = _NW * _PPW
_NZ = _NPAD - _N
_CHUNK = 4
_NCHUNK = _PPW // _CHUNK
_ROWS = _NOFF * _CHUNK
_ROWS_PAD = 112
_TBL = 39312
_L = 16
_NBUF = 4


def _rne16(x):
    xi = lax.bitcast_convert_type(x, jnp.int32)
    return lax.shift_right_logical(xi + 0x8000, 16)


def _mm_body(f_ref, w_ref, y_ref):
    f = f_ref[...]
    for o in range(_NOFF):
        y = jnp.dot(f, w_ref[o], preferred_element_type=jnp.float32)
        lo = _rne16(y[:, : _C // 2])
        hi = _rne16(y[:, _C // 2 :])
        y_ref[o] = jnp.bitwise_or(lo, lax.shift_left(hi, 16))


def _compute_y(fpad, w27):
    mtile = 256
    return pl.pallas_call(
        _mm_body,
        grid=(_NPAD // mtile,),
        in_specs=[
            pl.BlockSpec((mtile, _C), lambda i: (i, 0)),
            pl.BlockSpec((_NOFF, _C, _C), lambda i: (0, 0, 0)),
        ],
        out_specs=pl.BlockSpec((_NOFF, mtile, _C // 2), lambda i: (0, i, 0)),
        out_shape=jax.ShapeDtypeStruct((_NOFF, _NPAD, _C // 2), jnp.int32),
    )(fpad, w27)


_DELTAS = []
for _dx in range(-1, 2):
    for _dy in range(-1, 2):
        for _dz in range(-1, 2):
            _DELTAS.append(_dx * (_GP * _GP) + _dy * _GP + _dz)


def _sc_build_idx(lin_pad):
    mesh = plsc.VectorSubcoreMesh(core_axis_name="c", subcore_axis_name="s")

    @functools.partial(
        pl.kernel,
        mesh=mesh,
        out_type=jax.ShapeDtypeStruct((_NW, _NCHUNK * _ROWS_PAD), jnp.int32),
        compiler_params=pltpu.CompilerParams(needs_layout_passes=False),
        scratch_types=[
            pltpu.VMEM((_TBL,), jnp.int32),
            pltpu.VMEM((_NPAD,), jnp.int32),
            pltpu.VMEM((_NCHUNK * _ROWS_PAD,), jnp.int32),
        ],
    )
    def ka(lin_hbm, idx_hbm, tbl, lin_v, idx_v):
        cid = lax.axis_index("c")
        sid = lax.axis_index("s")
        wid = sid * 2 + cid
        base = wid * _PPW
        pltpu.sync_copy(lin_hbm, lin_v)

        lane = lax.iota(jnp.int32, _L)

        def zb(i, _):
            tbl[pl.ds(i * _L, _L)] = jnp.full((_L,), -1, jnp.int32)
            return 0

        lax.fori_loop(0, _TBL // _L, zb, 0)

        def sb(q, _):
            lv = lin_v[pl.ds(q * _L, _L)]
            ids = q * _L + lane
            plsc.store_scatter(tbl, [lv], ids)
            return 0

        lax.fori_loop(0, _NPAD // _L, sb, 0)

        def gb(g, _):
            pid = g * _L + lane
            lv = lin_v[pl.ds(base + g * _L, _L)]
            ci = lax.shift_right_logical(pid, 2)
            prem = jnp.bitwise_and(pid, 3)
            pos_base = ci * _ROWS_PAD + prem
            for o in range(_NOFF):
                nl = jnp.maximum(lv + _DELTAS[o], 0)
                t = plsc.load_gather(tbl, [nl])
                spread = o * _NPAD + _N + jnp.remainder(pid + o * 9, _NZ)
                gi = jnp.where(t >= 0, o * _NPAD + t, spread)
                plsc.store_scatter(idx_v, [pos_base + o * _CHUNK], gi)
            return 0

        lax.fori_loop(0, _PPW // _L, gb, 0)

        def pb(f, _):
            cis = f * _L + lane
            for kk in range(_ROWS_PAD - _ROWS):
                pos = cis * _ROWS_PAD + _ROWS + kk
                val = kk * _NPAD + _N + jnp.remainder(cis * 4 + kk * 61, _NZ)
                plsc.store_scatter(idx_v, [pos], val)
            return 0

        lax.fori_loop(0, _NCHUNK // _L, pb, 0)
        pltpu.sync_copy(idx_v, idx_hbm.at[wid])

    return ka(lin_pad)


def _sc_gather_reduce(y_flat, idx):
    mesh = plsc.VectorSubcoreMesh(core_axis_name="c", subcore_axis_name="s")

    @functools.partial(
        pl.kernel,
        mesh=mesh,
        out_type=jax.ShapeDtypeStruct((_NPAD, _C), jnp.float32),
        compiler_params=pltpu.CompilerParams(needs_layout_passes=False),
        scratch_types=[
            pltpu.VMEM((_NCHUNK * _ROWS_PAD,), jnp.int32),
            pltpu.VMEM((_NBUF, _ROWS_PAD, _C // 2), jnp.int32),
            pltpu.VMEM((_NBUF, _CHUNK, _C), jnp.float32),
            pltpu.SemaphoreType.DMA((_NBUF,)),
            pltpu.SemaphoreType.DMA((_NBUF,)),
        ],
    )
    def k(y_hbm, idx_hbm, out_hbm, idx_v, stag, acc, gsem, osem):
        cid = lax.axis_index("c")
        sid = lax.axis_index("s")
        wid = sid * 2 + cid
        base = wid * _PPW
        pltpu.sync_copy(idx_hbm.at[wid], idx_v)

        def fire(ci, b):
            pltpu.async_copy(
                y_hbm.at[idx_v.at[pl.ds(ci * _ROWS_PAD, _ROWS_PAD)]],
                stag.at[b],
                gsem.at[b],
            )

        def drain_gather(b):
            pltpu.make_async_copy(
                y_hbm.at[pl.ds(0, _ROWS_PAD)], stag.at[b], gsem.at[b]
            ).wait()

        def reduce(b):
            def cbody(c, _):
                def pbody(p, _):
                    ae, ao = plsc.unpack(
                        plsc.bitcast(
                            stag[b, p, pl.ds(c * _L, _L)], jnp.bfloat16
                        ),
                        format=plsc.PackFormat.INTERLEAVED,
                    )
                    for o in range(1, _NOFF):
                        e2, o2 = plsc.unpack(
                            plsc.bitcast(
                                stag[b, o * _CHUNK + p, pl.ds(c * _L, _L)],
                                jnp.bfloat16,
                            ),
                            format=plsc.PackFormat.INTERLEAVED,
                        )
                        ae = ae + e2
                        ao = ao + o2
                    acc[b, p, pl.ds(c * _L, _L)] = ae
                    acc[b, p, pl.ds(_C // 2 + c * _L, _L)] = ao
                    return 0

                lax.fori_loop(0, _CHUNK, pbody, 0)
                return 0

            lax.fori_loop(0, _C // (2 * _L), cbody, 0)

        for b in range(_NBUF):
            fire(b, b)

        def outer(t, _):
            for b in range(_NBUF):
                ci = t * _NBUF + b
                drain_gather(b)

                @pl.when(t > 0)
                def _drain_out():
                    pltpu.make_async_copy(
                        y_hbm.at[pl.ds(0, _CHUNK)], acc.at[b], osem.at[b]
                    ).wait()

                reduce(b)

                @pl.when(t < _NCHUNK // _NBUF - 1)
                def _fire_next():
                    fire(ci + _NBUF, b)

                pltpu.async_copy(
                    acc.at[b],
                    out_hbm.at[pl.ds(base + ci * _CHUNK, _CHUNK)],
                    osem.at[b],
                )
            return 0

        lax.fori_loop(0, _NCHUNK // _NBUF, outer, 0)
        for b in range(_NBUF):
            pltpu.make_async_copy(
                y_hbm.at[pl.ds(0, _CHUNK)], acc.at[b], osem.at[b]
            ).wait()

    return k(y_flat, idx)


def kernel(features, inp_positions, W, voxel_size):
    v = jnp.floor(inp_positions / voxel_size).astype(jnp.int32)
    lin = (v[:, 0] + 1) * (_GP * _GP) + (v[:, 1] + 1) * _GP + (v[:, 2] + 1)
    lin_pad = jnp.pad(lin, (0, _NPAD - _N))

    fpad = jnp.pad(features, ((0, _NPAD - _N), (0, 0))).astype(jnp.bfloat16)
    w27 = W.reshape(_NOFF, _C, _C).astype(jnp.bfloat16)
    idx = _sc_build_idx(lin_pad)
    y = _compute_y(fpad, w27).reshape(_NOFF * _NPAD, _C // 2)
    out = _sc_gather_reduce(y, idx)
    return out[:_N]


# speedup vs baseline: 9.6972x; 1.1414x over previous
import functools

import jax
import jax.numpy as jnp
from jax import lax
from jax.experimental import pallas as pl
from jax.experimental.pallas import tpu as pltpu
from jax.experimental.pallas import tpu_sc as plsc

_N = 10000
_G = 32
_GP = _G + 2
_C = 256
_NOFF = 27
_NW = 32
_PPW = 320
_NPAD = _NW * _PPW
_NZ = _NPAD - _N
_CHUNK = 4
_NCHUNK = _PPW // _CHUNK
_ROWS = _NOFF * _CHUNK
_ROWS_PAD = 112
_TBL = 39312
_L = 16
_NBUF = 5


def _rne16(x):
    xi = lax.bitcast_convert_type(x, jnp.int32)
    return lax.shift_right_logical(xi + 0x8000, 16)


def _mm_body(f_ref, w_ref, y_ref):
    f = f_ref[...]
    for o in range(_NOFF):
        y = jnp.dot(f, w_ref[o], preferred_element_type=jnp.float32)
        lo = _rne16(y[:, : _C // 2])
        hi = _rne16(y[:, _C // 2 :])
        y_ref[o] = jnp.bitwise_or(lo, lax.shift_left(hi, 16))


def _compute_y(fpad, w27):
    mtile = 512
    return pl.pallas_call(
        _mm_body,
        grid=(_NPAD // mtile,),
        in_specs=[
            pl.BlockSpec((mtile, _C), lambda i: (i, 0)),
            pl.BlockSpec((_NOFF, _C, _C), lambda i: (0, 0, 0)),
        ],
        out_specs=pl.BlockSpec((_NOFF, mtile, _C // 2), lambda i: (0, i, 0)),
        out_shape=jax.ShapeDtypeStruct((_NOFF, _NPAD, _C // 2), jnp.int32),
    )(fpad, w27)


_DELTAS = []
for _dx in range(-1, 2):
    for _dy in range(-1, 2):
        for _dz in range(-1, 2):
            _DELTAS.append(_dx * (_GP * _GP) + _dy * _GP + _dz)


def _sc_build_idx(lin_pad):
    mesh = plsc.VectorSubcoreMesh(core_axis_name="c", subcore_axis_name="s")

    @functools.partial(
        pl.kernel,
        mesh=mesh,
        out_type=jax.ShapeDtypeStruct((_NW, _NCHUNK * _ROWS_PAD), jnp.int32),
        compiler_params=pltpu.CompilerParams(needs_layout_passes=False),
        scratch_types=[
            pltpu.VMEM((_TBL,), jnp.int32),
            pltpu.VMEM((_NPAD,), jnp.int32),
            pltpu.VMEM((_NCHUNK * _ROWS_PAD,), jnp.int32),
        ],
    )
    def ka(lin_hbm, idx_hbm, tbl, lin_v, idx_v):
        cid = lax.axis_index("c")
        sid = lax.axis_index("s")
        wid = sid * 2 + cid
        base = wid * _PPW
        pltpu.sync_copy(lin_hbm, lin_v)

        lane = lax.iota(jnp.int32, _L)

        def zb(i, _):
            tbl[pl.ds(i * _L, _L)] = jnp.full((_L,), -1, jnp.int32)
            return 0

        lax.fori_loop(0, _TBL // _L, zb, 0)

        def sb(q, _):
            lv = lin_v[pl.ds(q * _L, _L)]
            ids = q * _L + lane
            plsc.store_scatter(tbl, [lv], ids)
            return 0

        lax.fori_loop(0, _NPAD // _L, sb, 0)

        def gb(g, _):
            pid = g * _L + lane
            lv = lin_v[pl.ds(base + g * _L, _L)]
            ci = lax.shift_right_logical(pid, 2)
            prem = jnp.bitwise_and(pid, 3)
            pos_base = ci * _ROWS_PAD + prem
            for o in range(_NOFF):
                nl = jnp.maximum(lv + _DELTAS[o], 0)
                t = plsc.load_gather(tbl, [nl])
                spread = o * _NPAD + _N + jnp.remainder(pid + o * 9, _NZ)
                gi = jnp.where(t >= 0, o * _NPAD + t, spread)
                plsc.store_scatter(idx_v, [pos_base + o * _CHUNK], gi)
            return 0

        lax.fori_loop(0, _PPW // _L, gb, 0)

        def pb(f, _):
            cis = f * _L + lane
            for kk in range(_ROWS_PAD - _ROWS):
                pos = cis * _ROWS_PAD + _ROWS + kk
                val = kk * _NPAD + _N + jnp.remainder(cis * 4 + kk * 61, _NZ)
                plsc.store_scatter(idx_v, [pos], val)
            return 0

        lax.fori_loop(0, _NCHUNK // _L, pb, 0)
        pltpu.sync_copy(idx_v, idx_hbm.at[wid])

    return ka(lin_pad)


def _sc_gather_reduce(y_flat, idx):
    mesh = plsc.VectorSubcoreMesh(core_axis_name="c", subcore_axis_name="s")

    @functools.partial(
        pl.kernel,
        mesh=mesh,
        out_type=jax.ShapeDtypeStruct((_NPAD, _C), jnp.float32),
        compiler_params=pltpu.CompilerParams(needs_layout_passes=False),
        scratch_types=[
            pltpu.VMEM((_NCHUNK * _ROWS_PAD,), jnp.int32),
            pltpu.VMEM((_NBUF, _ROWS_PAD, _C // 2), jnp.int32),
            pltpu.VMEM((_NBUF, _CHUNK, _C), jnp.float32),
            pltpu.SemaphoreType.DMA((_NBUF,)),
            pltpu.SemaphoreType.DMA((_NBUF,)),
        ],
    )
    def k(y_hbm, idx_hbm, out_hbm, idx_v, stag, acc, gsem, osem):
        cid = lax.axis_index("c")
        sid = lax.axis_index("s")
        wid = sid * 2 + cid
        base = wid * _PPW
        pltpu.sync_copy(idx_hbm.at[wid], idx_v)

        def fire(ci, b):
            pltpu.async_copy(
                y_hbm.at[idx_v.at[pl.ds(ci * _ROWS_PAD, _ROWS_PAD)]],
                stag.at[b],
                gsem.at[b],
            )

        def drain_gather(b):
            pltpu.make_async_copy(
                y_hbm.at[pl.ds(0, _ROWS_PAD)], stag.at[b], gsem.at[b]
            ).wait()

        def reduce(b):
            def cbody(c, _):
                def pbody(p, _):
                    vals = [
                        plsc.bitcast(
                            stag[b, o * _CHUNK + p, pl.ds(c * _L, _L)],
                            jnp.bfloat16,
                        )
                        for o in range(_NOFF)
                    ]
                    while len(vals) > 1:
                        nxt = [
                            vals[i] + vals[i + 1]
                            for i in range(0, len(vals) - 1, 2)
                        ]
                        if len(vals) % 2:
                            nxt.append(vals[-1])
                        vals = nxt
                    ae, ao = plsc.unpack(
                        vals[0], format=plsc.PackFormat.INTERLEAVED
                    )
                    acc[b, p, pl.ds(c * _L, _L)] = ae
                    acc[b, p, pl.ds(_C // 2 + c * _L, _L)] = ao
                    return 0

                lax.fori_loop(0, _CHUNK, pbody, 0)
                return 0

            lax.fori_loop(0, _C // (2 * _L), cbody, 0)

        for b in range(_NBUF):
            fire(b, b)

        def outer(t, _):
            for b in range(_NBUF):
                ci = t * _NBUF + b
                drain_gather(b)

                @pl.when(t > 0)
                def _drain_out():
                    pltpu.make_async_copy(
                        y_hbm.at[pl.ds(0, _CHUNK)], acc.at[b], osem.at[b]
                    ).wait()

                reduce(b)

                @pl.when(t < _NCHUNK // _NBUF - 1)
                def _fire_next():
                    fire(ci + _NBUF, b)

                pltpu.async_copy(
                    acc.at[b],
                    out_hbm.at[pl.ds(base + ci * _CHUNK, _CHUNK)],
                    osem.at[b],
                )
            return 0

        lax.fori_loop(0, _NCHUNK // _NBUF, outer, 0)
        for b in range(_NBUF):
            pltpu.make_async_copy(
                y_hbm.at[pl.ds(0, _CHUNK)], acc.at[b], osem.at[b]
            ).wait()

    return k(y_flat, idx)


def kernel(features, inp_positions, W, voxel_size):
    v = jnp.floor(inp_positions / voxel_size).astype(jnp.int32)
    lin = (v[:, 0] + 1) * (_GP * _GP) + (v[:, 1] + 1) * _GP + (v[:, 2] + 1)
    lin_pad = jnp.pad(lin, (0, _NPAD - _N))

    fpad = jnp.pad(features, ((0, _NPAD - _N), (0, 0))).astype(jnp.bfloat16)
    w27 = W.reshape(_NOFF, _C, _C).astype(jnp.bfloat16)
    idx = _sc_build_idx(lin_pad)
    y = _compute_y(fpad, w27).reshape(_NOFF * _NPAD, _C // 2)
    out = _sc_gather_reduce(y, idx)
    return out[:_N]
